# Initial kernel scaffold; baseline (speedup 1.0000x reference)
import jax
import jax.numpy as jnp
from jax.experimental import pallas as pl


def kernel(inputs, Wg, W1, W3, W2):
    raise NotImplementedError("write your pallas kernel here")


# speedup vs baseline: 2.6560x; 2.6560x over previous
import functools

import jax
import jax.numpy as jnp
from jax.experimental import pallas as pl

NUM_EXPERTS = 8
TOP_K = 2
D = 1024
DFF = 1024
BT = 512


def _dense_body(x_ref, wg_ref, w1_ref, w3_ref, w2_ref,
                ans_ref, f_ref, p_ref, load_ref, *, nt, ne, t_total):
    tb = pl.program_id(0)
    e = pl.program_id(1)

    x = x_ref[...]
    logits = jnp.dot(x, wg_ref[...], preferred_element_type=jnp.float32)
    m = jnp.max(logits, axis=1, keepdims=True)
    ex = jnp.exp(logits - m)
    probs = ex / jnp.sum(ex, axis=1, keepdims=True)
    iota = jax.lax.broadcasted_iota(jnp.int32, probs.shape, 1)
    v1 = jnp.max(probs, axis=1, keepdims=True)
    i1 = jnp.min(jnp.where(probs >= v1, iota, ne), axis=1, keepdims=True)
    p2 = jnp.where(iota == i1, -1.0, probs)
    v2 = jnp.max(p2, axis=1, keepdims=True)
    i2 = jnp.min(jnp.where(p2 >= v2, iota, ne), axis=1, keepdims=True)

    w_col = (jnp.where(i1 == e, v1, 0.0) + jnp.where(i2 == e, v2, 0.0))

    xn = x * jax.lax.rsqrt(jnp.mean(x * x, axis=1, keepdims=True) + 1e-8)

    h1 = jnp.dot(xn, w1_ref[0], preferred_element_type=jnp.float32)
    h3 = jnp.dot(xn, w3_ref[0], preferred_element_type=jnp.float32)
    h = (h1 * jax.nn.sigmoid(h1)) * h3
    o = jnp.dot(h, w2_ref[0], preferred_element_type=jnp.float32)
    contrib = o * w_col

    @pl.when(e == 0)
    def _init_ans():
        ans_ref[...] = contrib

    @pl.when(e != 0)
    def _acc_ans():
        ans_ref[...] += contrib

    @pl.when((tb == 0) & (e == 0))
    def _init_fp():
        f_ref[...] = jnp.zeros_like(f_ref)
        p_ref[...] = jnp.zeros_like(p_ref)

    @pl.when(e == 0)
    def _acc_fp():
        router = (iota == i1).astype(jnp.float32) + (iota == i2).astype(jnp.float32)
        f_ref[...] += jnp.sum(router, axis=0, keepdims=True)
        p_ref[...] += jnp.sum(probs, axis=0, keepdims=True)

    @pl.when((tb == nt - 1) & (e == ne - 1))
    def _final_load():
        load_ref[...] = (ne / (t_total * t_total)) * jnp.sum(
            f_ref[...] * p_ref[...], axis=1, keepdims=True)


def kernel(inputs, Wg, W1, W3, W2):
    bs, sl, d = inputs.shape
    t = bs * sl
    x = inputs.reshape(t, d)
    nt = t // BT
    ne = NUM_EXPERTS

    body = functools.partial(_dense_body, nt=nt, ne=ne, t_total=t)
    ans, fs, ps, load = pl.pallas_call(
        body,
        grid=(nt, ne),
        in_specs=[
            pl.BlockSpec((BT, d), lambda tb, e: (tb, 0)),
            pl.BlockSpec((d, ne), lambda tb, e: (0, 0)),
            pl.BlockSpec((1, d, DFF), lambda tb, e: (e, 0, 0)),
            pl.BlockSpec((1, d, DFF), lambda tb, e: (e, 0, 0)),
            pl.BlockSpec((1, DFF, d), lambda tb, e: (e, 0, 0)),
        ],
        out_specs=[
            pl.BlockSpec((BT, d), lambda tb, e: (tb, 0)),
            pl.BlockSpec((1, ne), lambda tb, e: (0, 0)),
            pl.BlockSpec((1, ne), lambda tb, e: (0, 0)),
            pl.BlockSpec((1, 1), lambda tb, e: (0, 0)),
        ],
        out_shape=[
            jax.ShapeDtypeStruct((t, d), jnp.float32),
            jax.ShapeDtypeStruct((1, ne), jnp.float32),
            jax.ShapeDtypeStruct((1, ne), jnp.float32),
            jax.ShapeDtypeStruct((1, 1), jnp.float32),
        ],
    )(x, Wg, W1, W3, W2)
    return ans.reshape(bs, sl, d), load[0, 0]


# speedup vs baseline: 2.6624x; 1.0024x over previous
import functools

import jax
import jax.numpy as jnp
from jax.experimental import pallas as pl

NUM_EXPERTS = 8
TOP_K = 2
D = 1024
DFF = 1024
BT = 512


def _dense_body(x_ref, wg_ref, w1_ref, w3_ref, w2_ref,
                ans_ref, f_ref, p_ref, load_ref, *, nt, ne, t_total):
    tb = pl.program_id(0)
    e = pl.program_id(1)

    x = x_ref[...]
    logits = jnp.dot(x, wg_ref[...], preferred_element_type=jnp.float32)
    m = jnp.max(logits, axis=1, keepdims=True)
    ex = jnp.exp(logits - m)
    probs = ex / jnp.sum(ex, axis=1, keepdims=True)
    iota = jax.lax.broadcasted_iota(jnp.int32, probs.shape, 1)
    v1 = jnp.max(probs, axis=1, keepdims=True)
    i1 = jnp.min(jnp.where(probs >= v1, iota, ne), axis=1, keepdims=True)
    p2 = jnp.where(iota == i1, -1.0, probs)
    v2 = jnp.max(p2, axis=1, keepdims=True)
    i2 = jnp.min(jnp.where(p2 >= v2, iota, ne), axis=1, keepdims=True)

    w_col = (jnp.where(i1 == e, v1, 0.0) + jnp.where(i2 == e, v2, 0.0))

    xn = x * jax.lax.rsqrt(jnp.mean(x * x, axis=1, keepdims=True) + 1e-8)

    xnb = xn.astype(jnp.bfloat16)
    h1 = jnp.dot(xnb, w1_ref[0].astype(jnp.bfloat16),
                 preferred_element_type=jnp.float32)
    h3 = jnp.dot(xnb, w3_ref[0].astype(jnp.bfloat16),
                 preferred_element_type=jnp.float32)
    h = (h1 * jax.nn.sigmoid(h1)) * h3
    o = jnp.dot(h.astype(jnp.bfloat16), w2_ref[0].astype(jnp.bfloat16),
                preferred_element_type=jnp.float32)
    contrib = o * w_col

    @pl.when(e == 0)
    def _init_ans():
        ans_ref[...] = contrib

    @pl.when(e != 0)
    def _acc_ans():
        ans_ref[...] += contrib

    @pl.when((tb == 0) & (e == 0))
    def _init_fp():
        f_ref[...] = jnp.zeros_like(f_ref)
        p_ref[...] = jnp.zeros_like(p_ref)

    @pl.when(e == 0)
    def _acc_fp():
        router = (iota == i1).astype(jnp.float32) + (iota == i2).astype(jnp.float32)
        f_ref[...] += jnp.sum(router, axis=0, keepdims=True)
        p_ref[...] += jnp.sum(probs, axis=0, keepdims=True)

    @pl.when((tb == nt - 1) & (e == ne - 1))
    def _final_load():
        load_ref[...] = (ne / (t_total * t_total)) * jnp.sum(
            f_ref[...] * p_ref[...], axis=1, keepdims=True)


def kernel(inputs, Wg, W1, W3, W2):
    bs, sl, d = inputs.shape
    t = bs * sl
    x = inputs.reshape(t, d)
    nt = t // BT
    ne = NUM_EXPERTS

    body = functools.partial(_dense_body, nt=nt, ne=ne, t_total=t)
    ans, fs, ps, load = pl.pallas_call(
        body,
        grid=(nt, ne),
        in_specs=[
            pl.BlockSpec((BT, d), lambda tb, e: (tb, 0)),
            pl.BlockSpec((d, ne), lambda tb, e: (0, 0)),
            pl.BlockSpec((1, d, DFF), lambda tb, e: (e, 0, 0)),
            pl.BlockSpec((1, d, DFF), lambda tb, e: (e, 0, 0)),
            pl.BlockSpec((1, DFF, d), lambda tb, e: (e, 0, 0)),
        ],
        out_specs=[
            pl.BlockSpec((BT, d), lambda tb, e: (tb, 0)),
            pl.BlockSpec((1, ne), lambda tb, e: (0, 0)),
            pl.BlockSpec((1, ne), lambda tb, e: (0, 0)),
            pl.BlockSpec((1, 1), lambda tb, e: (0, 0)),
        ],
        out_shape=[
            jax.ShapeDtypeStruct((t, d), jnp.float32),
            jax.ShapeDtypeStruct((1, ne), jnp.float32),
            jax.ShapeDtypeStruct((1, ne), jnp.float32),
            jax.ShapeDtypeStruct((1, 1), jnp.float32),
        ],
    )(x, Wg, W1, W3, W2)
    return ans.reshape(bs, sl, d), load[0, 0]
